# baseline (speedup 1.0000x reference)
import functools

import jax
import jax.numpy as jnp
import numpy as np
from jax import lax
from jax.experimental import pallas as pl
from jax.experimental.pallas import tpu as pltpu
from jax.experimental.pallas import tpu_sc as plsc

N = 10000
E = 320000
F = 128
HEADS = 16
CH = 8
NC = 2
NS = 16
NW = NC * NS
EPW = E // NW
K = 40
NCH_E = EPW // K
NCH_N = N // K

_j = np.arange(F)
_PERM = (_j % HEADS) * CH + (_j // HEADS)

_ROW_BLK = 1000
_GRID = N // _ROW_BLK


def _perm_np():
    return _PERM


def _proj_body(x_ref, wl_ref, wr_ref, bl_ref, br_ref, xl_ref, xr_ref):
    xb = x_ref[...]
    xl_ref[...] = lax.dot_general(
        xb, wl_ref[...], (((1,), (1,)), ((), ())),
        preferred_element_type=jnp.float32) + bl_ref[...]
    xr_ref[...] = lax.dot_general(
        xb, wr_ref[...], (((1,), (1,)), ((), ())),
        preferred_element_type=jnp.float32) + br_ref[...]


def _tc_proj(x, wl, wr, bl, br):
    return pl.pallas_call(
        _proj_body,
        grid=(_GRID,),
        in_specs=[
            pl.BlockSpec((_ROW_BLK, F), lambda i: (i, 0)),
            pl.BlockSpec((F, F), lambda i: (0, 0)),
            pl.BlockSpec((F, F), lambda i: (0, 0)),
            pl.BlockSpec((1, F), lambda i: (0, 0)),
            pl.BlockSpec((1, F), lambda i: (0, 0)),
        ],
        out_specs=[
            pl.BlockSpec((_ROW_BLK, F), lambda i: (i, 0)),
            pl.BlockSpec((_ROW_BLK, F), lambda i: (i, 0)),
        ],
        out_shape=[
            jax.ShapeDtypeStruct((N, F), jnp.float32),
            jax.ShapeDtypeStruct((N, F), jnp.float32),
        ],
    )(x, wl, wr, bl, br)


def _mid_body(acc_ref, den_ref, t_ref, b1_ref, wl_ref, wr_ref, bl_ref,
              br_ref, xl_ref, xr_ref):
    a = acc_ref[0] + acc_ref[1]
    d = den_ref[0] + den_ref[1]
    denb = lax.dot_general(d, t_ref[...], (((1,), (0,)), ((), ())),
                           preferred_element_type=jnp.float32)
    h = a / (denb + 1e-16) + b1_ref[...]
    h = jnp.maximum(h, 0.0) + (jnp.exp(jnp.minimum(h, 0.0)) - 1.0)
    xl_ref[...] = lax.dot_general(
        h, wl_ref[...], (((1,), (1,)), ((), ())),
        preferred_element_type=jnp.float32) + bl_ref[...]
    xr_ref[...] = lax.dot_general(
        h, wr_ref[...], (((1,), (1,)), ((), ())),
        preferred_element_type=jnp.float32) + br_ref[...]


def _tc_mid(acc2, den2, t, b1, wl, wr, bl, br):
    return pl.pallas_call(
        _mid_body,
        grid=(_GRID,),
        in_specs=[
            pl.BlockSpec((NC, _ROW_BLK, F), lambda i: (0, i, 0)),
            pl.BlockSpec((NC, _ROW_BLK, HEADS), lambda i: (0, i, 0)),
            pl.BlockSpec((HEADS, F), lambda i: (0, 0)),
            pl.BlockSpec((1, F), lambda i: (0, 0)),
            pl.BlockSpec((F, F), lambda i: (0, 0)),
            pl.BlockSpec((F, F), lambda i: (0, 0)),
            pl.BlockSpec((1, F), lambda i: (0, 0)),
            pl.BlockSpec((1, F), lambda i: (0, 0)),
        ],
        out_specs=[
            pl.BlockSpec((_ROW_BLK, F), lambda i: (i, 0)),
            pl.BlockSpec((_ROW_BLK, F), lambda i: (i, 0)),
        ],
        out_shape=[
            jax.ShapeDtypeStruct((N, F), jnp.float32),
            jax.ShapeDtypeStruct((N, F), jnp.float32),
        ],
    )(acc2, den2, t, b1, wl, wr, bl, br)


def _final_body(acc_ref, den_ref, t_ref, b2_ref, p_ref, out_ref):
    a = acc_ref[0] + acc_ref[1]
    d = den_ref[0] + den_ref[1]
    denb = lax.dot_general(d, t_ref[...], (((1,), (0,)), ((), ())),
                           preferred_element_type=jnp.float32)
    o = a / (denb + 1e-16) + b2_ref[...]
    o = o - jnp.max(o, axis=-1, keepdims=True)
    o = o - jnp.log(jnp.sum(jnp.exp(o), axis=-1, keepdims=True))
    out_ref[...] = lax.dot_general(o, p_ref[...], (((1,), (0,)), ((), ())),
                                   preferred_element_type=jnp.float32)


def _tc_final(acc2, den2, t, b2, p):
    return pl.pallas_call(
        _final_body,
        grid=(_GRID,),
        in_specs=[
            pl.BlockSpec((NC, _ROW_BLK, F), lambda i: (0, i, 0)),
            pl.BlockSpec((NC, _ROW_BLK, HEADS), lambda i: (0, i, 0)),
            pl.BlockSpec((HEADS, F), lambda i: (0, 0)),
            pl.BlockSpec((1, F), lambda i: (0, 0)),
            pl.BlockSpec((F, F), lambda i: (0, 0)),
        ],
        out_specs=pl.BlockSpec((_ROW_BLK, F), lambda i: (i, 0)),
        out_shape=jax.ShapeDtypeStruct((N, F), jnp.float32),
    )(acc2, den2, t, b2, p)


SROWS = 11280
NCH_Z = SROWS // K
KP = K + HEADS
NPAIR = NCH_E // 2


def _sc_edge_body(xl_hbm, xr_hbm, att_hbm, src_hbm, dst_hbm, out_hbm,
         ad_s,
         srcb0, srcb1, dstb0, dstb1, didxb0, didxb1, sdst0, sdst1,
         xlb0, xlb1, xrb0, xrb1, msgb0, msgb1, denb, attb,
         gl0, gl1, gr0, gr1, ms0, ms1):
    cid = lax.axis_index("c")
    sid = lax.axis_index("s")
    wid = cid * NS + sid
    base = wid * EPW

    pltpu.sync_copy(att_hbm, attb)
    att_vs = [attb[pl.ds(c * HEADS, HEADS)] for c in range(CH)]
    zero16 = jnp.zeros((HEADS,), jnp.float32)

    def zrow(i, carry):
        for c in range(CH):
            xlb0[i, pl.ds(c * HEADS, HEADS)] = zero16
        return carry

    lax.fori_loop(0, K, zrow, 0)

    def zchunk(j, carry):
        ch = sid + NS * j

        @pl.when(ch < NCH_Z)
        def _():
            pltpu.sync_copy(xlb0, ad_s.at[pl.ds(ch * K, K)])
        return carry

    lax.fori_loop(0, (NCH_Z + NS - 1) // NS, zchunk, 0)
    plsc.subcore_barrier()

    bufs = ((srcb0, dstb0, didxb0, sdst0, xlb0, xrb0, msgb0, gl0, gr0, ms0),
            (srcb1, dstb1, didxb1, sdst1, xlb1, xrb1, msgb1, gl1, gr1, ms1))

    def issue_gather(s, c):
        srcb, dstb, didxb, _, xlb, xrb, _, gl, gr, _ = bufs[s]
        eoff = base + c * K
        pltpu.sync_copy(src_hbm.at[pl.ds(eoff, K)], srcb)
        pltpu.sync_copy(dst_hbm.at[pl.ds(eoff, K)], dstb.at[pl.ds(0, K)])
        pltpu.async_copy(xl_hbm.at[srcb], xlb, gl)
        pltpu.async_copy(xr_hbm.at[dstb.at[pl.ds(0, K)]], xrb, gr)
        for g in (0, 1):
            dv = dstb[pl.ds(g * HEADS, HEADS)]
            didxb[pl.ds(g * HEADS, HEADS)] = N + (dv >> 3)
        dv = dstb[pl.ds(K - HEADS, HEADS)]
        didxb[pl.ds(K - HEADS, HEADS)] = N + (dv >> 3)

    def wait_gather(s):
        srcb, dstb, _, _, xlb, xrb, _, gl, gr, _ = bufs[s]
        pltpu.make_async_copy(xl_hbm.at[srcb], xlb, gl).wait()
        pltpu.make_async_copy(xr_hbm.at[dstb.at[pl.ds(0, K)]], xrb, gr).wait()

    def wait_msg_scatter(s):
        _, _, _, sdst, _, _, msgb, _, _, ms = bufs[s]
        pltpu.make_async_copy(msgb, ad_s.at[sdst], ms).wait()

    def process(s, j):
        srcb, dstb, didxb, sdst, xlb, xrb, msgb, _, _, ms = bufs[s]
        wait_gather(s)

        @plsc.parallel_loop(0, K, unroll=16)
        def edge_body(i):
            xls = [xlb[i, pl.ds(c * HEADS, HEADS)] for c in range(CH)]
            alpha = None
            for c in range(CH):
                sv = xls[c] + xrb[i, pl.ds(c * HEADS, HEADS)]
                ev = jnp.maximum(sv, sv * 0.2)
                tv = ev * att_vs[c]
                alpha = tv if alpha is None else alpha + tv
            ex = jnp.exp(alpha)
            for c in range(CH):
                msgb[i, pl.ds(c * HEADS, HEADS)] = xls[c] * ex
            d = dstb[pl.ds(i, HEADS)][0]
            slot = jnp.bitwise_and(d, CH - 1) * HEADS
            for c in range(CH):
                denb[i, pl.ds(c * HEADS, HEADS)] = zero16
            denb[i, pl.ds(slot, HEADS)] = ex
        sdst[pl.ds(0, HEADS)] = dstb[pl.ds(0, HEADS)]
        sdst[pl.ds(HEADS, HEADS)] = dstb[pl.ds(HEADS, HEADS)]
        sdst[pl.ds(K - HEADS, HEADS)] = dstb[pl.ds(K - HEADS, HEADS)]
        pltpu.sync_copy(denb, ad_s.at[didxb], add=True)
        pltpu.async_copy(msgb, ad_s.at[sdst], ms, add=True)

    issue_gather(0, 0)

    def pair_body(j, carry):
        @pl.when(j > 0)
        def _():
            wait_msg_scatter(1)
        issue_gather(1, 2 * j + 1)

        @pl.when(j > 0)
        def _():
            wait_msg_scatter(0)
        process(0, j)

        @pl.when(j < NPAIR - 1)
        def _():
            issue_gather(0, 2 * j + 2)
        process(1, j)
        return carry

    lax.fori_loop(0, NPAIR, pair_body, 0)
    wait_msg_scatter(0)
    wait_msg_scatter(1)
    plsc.subcore_barrier()

    def out_body(j, carry):
        ch = sid + NS * j

        @pl.when(ch < NCH_Z)
        def _():
            sl = pl.ds(ch * K, K)
            pltpu.sync_copy(ad_s.at[sl], xlb0)
            pltpu.sync_copy(xlb0, out_hbm.at[pl.ds(cid * SROWS + ch * K, K)])
        return carry

    lax.fori_loop(0, (NCH_Z + NS - 1) // NS, out_body, 0)


@functools.lru_cache(maxsize=1)
def _build_sc_edge_pass():
  return pl.kernel(
    _sc_edge_body,
    out_type=jax.ShapeDtypeStruct((NC * SROWS, F), jnp.float32),
    mesh=plsc.VectorSubcoreMesh(core_axis_name="c", subcore_axis_name="s",
                                num_cores=NC, num_subcores=NS),
    scratch_types=[
        pltpu.VMEM_SHARED((SROWS, F), jnp.float32),
        pltpu.VMEM((K,), jnp.int32), pltpu.VMEM((K,), jnp.int32),
        pltpu.VMEM((KP,), jnp.int32), pltpu.VMEM((KP,), jnp.int32),
        pltpu.VMEM((K,), jnp.int32), pltpu.VMEM((K,), jnp.int32),
        pltpu.VMEM((K,), jnp.int32), pltpu.VMEM((K,), jnp.int32),
        pltpu.VMEM((K, F), jnp.float32), pltpu.VMEM((K, F), jnp.float32),
        pltpu.VMEM((K, F), jnp.float32), pltpu.VMEM((K, F), jnp.float32),
        pltpu.VMEM((K, F), jnp.float32), pltpu.VMEM((K, F), jnp.float32),
        pltpu.VMEM((K, F), jnp.float32),
        pltpu.VMEM((F,), jnp.float32),
        pltpu.SemaphoreType.DMA, pltpu.SemaphoreType.DMA,
        pltpu.SemaphoreType.DMA, pltpu.SemaphoreType.DMA,
        pltpu.SemaphoreType.DMA, pltpu.SemaphoreType.DMA,
    ],
  )


def kernel(x, edge_index, Wl1, bl1, Wr1, br1, att1, bias1,
           Wl2, bl2, Wr2, br2, att2, bias2):
    perm = _PERM
    src = edge_index[0]
    dst = edge_index[1]

    wl1p = Wl1[perm]
    wr1p = Wr1[perm]
    bl1p = bl1[perm].reshape(1, F)
    br1p = br1[perm].reshape(1, F)
    att1cm = att1[0].T.reshape(F)
    b1cm = bias1[perm].reshape(1, F)

    wl2p = Wl2[perm][:, perm]
    wr2p = Wr2[perm][:, perm]
    bl2p = bl2[perm].reshape(1, F)
    br2p = br2[perm].reshape(1, F)
    att2cm = att2[0].T.reshape(F)
    b2cm = bias2[perm].reshape(1, F)

    t_np = np.zeros((HEADS, F), np.float32)
    t_np[_j % HEADS, _j] = 1.0
    t_mat = jnp.asarray(t_np)
    p_np = np.zeros((F, F), np.float32)
    p_np[_j, perm] = 1.0
    p_mat = jnp.asarray(p_np)
    def unpack(r):
        rr = r.reshape(NC, SROWS, F)
        acc2 = rr[:, :N, :]
        den2 = rr[:, N:N + N // CH, :].reshape(NC, N, HEADS)
        return acc2, den2

    sc_edge_pass = _build_sc_edge_pass()
    xl1, xr1 = _tc_proj(x, wl1p, wr1p, bl1p, br1p)
    acc2, den2 = unpack(sc_edge_pass(xl1, xr1, att1cm, src, dst))
    xl2, xr2 = _tc_mid(acc2, den2, t_mat, b1cm, wl2p, wr2p, bl2p, br2p)
    acc2b, den2b = unpack(sc_edge_pass(xl2, xr2, att2cm, src, dst))
    return _tc_final(acc2b, den2b, t_mat, b2cm, p_mat)


# speedup vs baseline: 1.1196x; 1.1196x over previous
import functools

import jax
import jax.numpy as jnp
import numpy as np
from jax import lax
from jax.experimental import pallas as pl
from jax.experimental.pallas import tpu as pltpu
from jax.experimental.pallas import tpu_sc as plsc

N = 10000
E = 320000
F = 128
HEADS = 16
CH = 8
NC = 2
NS = 16
NW = NC * NS
EPW = E // NW
K = 40
NCH_E = EPW // K

_j = np.arange(F)
_PERM = (_j % HEADS) * CH + (_j // HEADS)

_ROW_BLK = 1000
_GRID = N // _ROW_BLK


def _perm_np():
    return _PERM


def _proj_body(x_ref, wl_ref, wr_ref, bl_ref, br_ref, xl_ref, xr_ref):
    xb = x_ref[...]
    xl_ref[...] = lax.dot_general(
        xb, wl_ref[...], (((1,), (1,)), ((), ())),
        preferred_element_type=jnp.float32) + bl_ref[...]
    xr_ref[...] = lax.dot_general(
        xb, wr_ref[...], (((1,), (1,)), ((), ())),
        preferred_element_type=jnp.float32) + br_ref[...]


def _tc_proj(x, wl, wr, bl, br):
    return pl.pallas_call(
        _proj_body,
        grid=(_GRID,),
        in_specs=[
            pl.BlockSpec((_ROW_BLK, F), lambda i: (i, 0)),
            pl.BlockSpec((F, F), lambda i: (0, 0)),
            pl.BlockSpec((F, F), lambda i: (0, 0)),
            pl.BlockSpec((1, F), lambda i: (0, 0)),
            pl.BlockSpec((1, F), lambda i: (0, 0)),
        ],
        out_specs=[
            pl.BlockSpec((_ROW_BLK, F), lambda i: (i, 0)),
            pl.BlockSpec((_ROW_BLK, F), lambda i: (i, 0)),
        ],
        out_shape=[
            jax.ShapeDtypeStruct((N, F), jnp.float32),
            jax.ShapeDtypeStruct((N, F), jnp.float32),
        ],
    )(x, wl, wr, bl, br)


def _mid_body(acc_ref, den_ref, t_ref, b1_ref, wl_ref, wr_ref, bl_ref,
              br_ref, xl_ref, xr_ref):
    a = acc_ref[0] + acc_ref[1]
    d = den_ref[0] + den_ref[1]
    denb = lax.dot_general(d, t_ref[...], (((1,), (0,)), ((), ())),
                           preferred_element_type=jnp.float32)
    h = a / (denb + 1e-16) + b1_ref[...]
    h = jnp.maximum(h, 0.0) + (jnp.exp(jnp.minimum(h, 0.0)) - 1.0)
    xl_ref[...] = lax.dot_general(
        h, wl_ref[...], (((1,), (1,)), ((), ())),
        preferred_element_type=jnp.float32) + bl_ref[...]
    xr_ref[...] = lax.dot_general(
        h, wr_ref[...], (((1,), (1,)), ((), ())),
        preferred_element_type=jnp.float32) + br_ref[...]


def _tc_mid(acc2, den2, t, b1, wl, wr, bl, br):
    return pl.pallas_call(
        _mid_body,
        grid=(_GRID,),
        in_specs=[
            pl.BlockSpec((NC, _ROW_BLK, F), lambda i: (0, i, 0)),
            pl.BlockSpec((NC, _ROW_BLK, HEADS), lambda i: (0, i, 0)),
            pl.BlockSpec((HEADS, F), lambda i: (0, 0)),
            pl.BlockSpec((1, F), lambda i: (0, 0)),
            pl.BlockSpec((F, F), lambda i: (0, 0)),
            pl.BlockSpec((F, F), lambda i: (0, 0)),
            pl.BlockSpec((1, F), lambda i: (0, 0)),
            pl.BlockSpec((1, F), lambda i: (0, 0)),
        ],
        out_specs=[
            pl.BlockSpec((_ROW_BLK, F), lambda i: (i, 0)),
            pl.BlockSpec((_ROW_BLK, F), lambda i: (i, 0)),
        ],
        out_shape=[
            jax.ShapeDtypeStruct((N, F), jnp.float32),
            jax.ShapeDtypeStruct((N, F), jnp.float32),
        ],
    )(acc2, den2, t, b1, wl, wr, bl, br)


def _final_body(acc_ref, den_ref, t_ref, b2_ref, p_ref, out_ref):
    a = acc_ref[0] + acc_ref[1]
    d = den_ref[0] + den_ref[1]
    denb = lax.dot_general(d, t_ref[...], (((1,), (0,)), ((), ())),
                           preferred_element_type=jnp.float32)
    o = a / (denb + 1e-16) + b2_ref[...]
    o = o - jnp.max(o, axis=-1, keepdims=True)
    o = o - jnp.log(jnp.sum(jnp.exp(o), axis=-1, keepdims=True))
    out_ref[...] = lax.dot_general(o, p_ref[...], (((1,), (0,)), ((), ())),
                                   preferred_element_type=jnp.float32)


def _tc_final(acc2, den2, t, b2, p):
    return pl.pallas_call(
        _final_body,
        grid=(_GRID,),
        in_specs=[
            pl.BlockSpec((NC, _ROW_BLK, F), lambda i: (0, i, 0)),
            pl.BlockSpec((NC, _ROW_BLK, HEADS), lambda i: (0, i, 0)),
            pl.BlockSpec((HEADS, F), lambda i: (0, 0)),
            pl.BlockSpec((1, F), lambda i: (0, 0)),
            pl.BlockSpec((F, F), lambda i: (0, 0)),
        ],
        out_specs=pl.BlockSpec((_ROW_BLK, F), lambda i: (i, 0)),
        out_shape=jax.ShapeDtypeStruct((N, F), jnp.float32),
    )(acc2, den2, t, b2, p)


SROWS = 11280
NCH_Z = SROWS // K
KP = K + HEADS
NPAIR = NCH_E // 2


def _sc_edge_body(xl_hbm, xr_hbm, att_hbm, src_hbm, dst_hbm, out_hbm,
         ad_s,
         srcb0, srcb1, dstb0, dstb1, didxb0, didxb1, sdst0, sdst1,
         xlb0, xlb1, xrb0, xrb1, msgb0, msgb1, denb, attb,
         gl0, gl1, gr0, gr1, ms0, ms1):
    cid = lax.axis_index("c")
    sid = lax.axis_index("s")
    wid = cid * NS + sid
    base = wid * EPW

    pltpu.sync_copy(att_hbm, attb)
    att_vs = [attb[pl.ds(c * HEADS, HEADS)] for c in range(CH)]
    zero16 = jnp.zeros((HEADS,), jnp.float32)

    def zrow(i, carry):
        for c in range(CH):
            xlb0[i, pl.ds(c * HEADS, HEADS)] = zero16
        return carry

    lax.fori_loop(0, K, zrow, 0)

    def zchunk(j, carry):
        ch = sid + NS * j

        @pl.when(ch < NCH_Z)
        def _():
            pltpu.sync_copy(xlb0, ad_s.at[pl.ds(ch * K, K)])
        return carry

    lax.fori_loop(0, (NCH_Z + NS - 1) // NS, zchunk, 0)
    plsc.subcore_barrier()

    bufs = ((srcb0, dstb0, didxb0, sdst0, xlb0, xrb0, msgb0, gl0, gr0, ms0),
            (srcb1, dstb1, didxb1, sdst1, xlb1, xrb1, msgb1, gl1, gr1, ms1))

    def issue_gather(s, c):
        srcb, dstb, didxb, _, xlb, xrb, _, gl, gr, _ = bufs[s]
        eoff = base + c * K
        pltpu.sync_copy(src_hbm.at[pl.ds(eoff, K)], srcb)
        pltpu.sync_copy(dst_hbm.at[pl.ds(eoff, K)], dstb.at[pl.ds(0, K)])
        pltpu.async_copy(xl_hbm.at[srcb], xlb, gl)
        pltpu.async_copy(xr_hbm.at[dstb.at[pl.ds(0, K)]], xrb, gr)
        for g in (0, 1):
            dv = dstb[pl.ds(g * HEADS, HEADS)]
            didxb[pl.ds(g * HEADS, HEADS)] = N + (dv >> 3)
        dv = dstb[pl.ds(K - HEADS, HEADS)]
        didxb[pl.ds(K - HEADS, HEADS)] = N + (dv >> 3)

    def wait_gather(s):
        srcb, dstb, _, _, xlb, xrb, _, gl, gr, _ = bufs[s]
        pltpu.make_async_copy(xl_hbm.at[srcb], xlb, gl).wait()
        pltpu.make_async_copy(xr_hbm.at[dstb.at[pl.ds(0, K)]], xrb, gr).wait()

    def wait_msg_scatter(s):
        _, _, _, sdst, _, _, msgb, _, _, ms = bufs[s]
        pltpu.make_async_copy(msgb, ad_s.at[sdst], ms).wait()

    def process(s, j):
        srcb, dstb, didxb, sdst, xlb, xrb, msgb, _, _, ms = bufs[s]
        wait_gather(s)

        @plsc.parallel_loop(0, K, unroll=8)
        def edge_body(i):
            xls = [xlb[i, pl.ds(c * HEADS, HEADS)] for c in range(CH)]
            alpha = None
            for c in range(CH):
                sv = xls[c] + xrb[i, pl.ds(c * HEADS, HEADS)]
                ev = jnp.maximum(sv, sv * 0.2)
                tv = ev * att_vs[c]
                alpha = tv if alpha is None else alpha + tv
            ex = jnp.exp(alpha)
            for c in range(CH):
                msgb[i, pl.ds(c * HEADS, HEADS)] = xls[c] * ex
            d = dstb[pl.ds(i, HEADS)][0]
            slot = jnp.bitwise_and(d, CH - 1) * HEADS
            for c in range(CH):
                denb[i, pl.ds(c * HEADS, HEADS)] = zero16
            denb[i, pl.ds(slot, HEADS)] = ex
        sdst[pl.ds(0, HEADS)] = dstb[pl.ds(0, HEADS)]
        sdst[pl.ds(HEADS, HEADS)] = dstb[pl.ds(HEADS, HEADS)]
        sdst[pl.ds(K - HEADS, HEADS)] = dstb[pl.ds(K - HEADS, HEADS)]
        pltpu.sync_copy(denb, ad_s.at[didxb], add=True)
        pltpu.async_copy(msgb, ad_s.at[sdst], ms, add=True)

    issue_gather(0, 0)

    def pair_body(j, carry):
        @pl.when(j > 0)
        def _():
            wait_msg_scatter(1)
        issue_gather(1, 2 * j + 1)

        @pl.when(j > 0)
        def _():
            wait_msg_scatter(0)
        process(0, j)

        @pl.when(j < NPAIR - 1)
        def _():
            issue_gather(0, 2 * j + 2)
        process(1, j)
        return carry

    lax.fori_loop(0, NPAIR, pair_body, 0)
    wait_msg_scatter(0)
    wait_msg_scatter(1)
    plsc.subcore_barrier()

    def out_body(j, carry):
        ch = sid + NS * j

        @pl.when(ch < NCH_Z)
        def _():
            sl = pl.ds(ch * K, K)
            pltpu.sync_copy(ad_s.at[sl], xlb0)
            pltpu.sync_copy(xlb0, out_hbm.at[pl.ds(cid * SROWS + ch * K, K)])
        return carry

    lax.fori_loop(0, (NCH_Z + NS - 1) // NS, out_body, 0)


@functools.lru_cache(maxsize=1)
def _build_sc_edge_pass():
  return pl.kernel(
    _sc_edge_body,
    out_type=jax.ShapeDtypeStruct((NC * SROWS, F), jnp.float32),
    mesh=plsc.VectorSubcoreMesh(core_axis_name="c", subcore_axis_name="s",
                                num_cores=NC, num_subcores=NS),
    scratch_types=[
        pltpu.VMEM_SHARED((SROWS, F), jnp.float32),
        pltpu.VMEM((K,), jnp.int32), pltpu.VMEM((K,), jnp.int32),
        pltpu.VMEM((KP,), jnp.int32), pltpu.VMEM((KP,), jnp.int32),
        pltpu.VMEM((K,), jnp.int32), pltpu.VMEM((K,), jnp.int32),
        pltpu.VMEM((K,), jnp.int32), pltpu.VMEM((K,), jnp.int32),
        pltpu.VMEM((K, F), jnp.float32), pltpu.VMEM((K, F), jnp.float32),
        pltpu.VMEM((K, F), jnp.float32), pltpu.VMEM((K, F), jnp.float32),
        pltpu.VMEM((K, F), jnp.float32), pltpu.VMEM((K, F), jnp.float32),
        pltpu.VMEM((K, F), jnp.float32),
        pltpu.VMEM((F,), jnp.float32),
        pltpu.SemaphoreType.DMA, pltpu.SemaphoreType.DMA,
        pltpu.SemaphoreType.DMA, pltpu.SemaphoreType.DMA,
        pltpu.SemaphoreType.DMA, pltpu.SemaphoreType.DMA,
    ],
  )


def kernel(x, edge_index, Wl1, bl1, Wr1, br1, att1, bias1,
           Wl2, bl2, Wr2, br2, att2, bias2):
    perm = _PERM
    src = edge_index[0]
    dst = edge_index[1]

    wl1p = Wl1[perm]
    wr1p = Wr1[perm]
    bl1p = bl1[perm].reshape(1, F)
    br1p = br1[perm].reshape(1, F)
    att1cm = att1[0].T.reshape(F)
    b1cm = bias1[perm].reshape(1, F)

    wl2p = Wl2[perm][:, perm]
    wr2p = Wr2[perm][:, perm]
    bl2p = bl2[perm].reshape(1, F)
    br2p = br2[perm].reshape(1, F)
    att2cm = att2[0].T.reshape(F)
    b2cm = bias2[perm].reshape(1, F)

    t_np = np.zeros((HEADS, F), np.float32)
    t_np[_j % HEADS, _j] = 1.0
    t_mat = jnp.asarray(t_np)
    p_np = np.zeros((F, F), np.float32)
    p_np[_j, perm] = 1.0
    p_mat = jnp.asarray(p_np)
    def unpack(r):
        rr = r.reshape(NC, SROWS, F)
        acc2 = rr[:, :N, :]
        den2 = rr[:, N:N + N // CH, :].reshape(NC, N, HEADS)
        return acc2, den2

    sc_edge_pass = _build_sc_edge_pass()
    xl1, xr1 = _tc_proj(x, wl1p, wr1p, bl1p, br1p)
    acc2, den2 = unpack(sc_edge_pass(xl1, xr1, att1cm, src, dst))
    xl2, xr2 = _tc_mid(acc2, den2, t_mat, b1cm, wl2p, wr2p, bl2p, br2p)
    acc2b, den2b = unpack(sc_edge_pass(xl2, xr2, att2cm, src, dst))
    return _tc_final(acc2b, den2b, t_mat, b2cm, p_mat)
